# Initial kernel scaffold; baseline (speedup 1.0000x reference)
import jax
import jax.numpy as jnp
from jax.experimental import pallas as pl


def kernel(seq_len, table):
    raise NotImplementedError("write your pallas kernel here")


# speedup vs baseline: 1.5745x; 1.5745x over previous
import functools

import jax
import jax.numpy as jnp
from jax import lax
from jax.experimental import pallas as pl
from jax.experimental.pallas import tpu as pltpu
from jax.experimental.pallas import tpu_sc as plsc

N = 8192
D = 1024
NC = 2
NS = 16
NW = NC * NS
R = N // NW
C = 32
NCH = R // C
NBUF = 3


def _make_gather():
    mesh = plsc.VectorSubcoreMesh(core_axis_name="c", subcore_axis_name="s")
    scratch = [pltpu.VMEM((NCH, C), jnp.int32)]
    scratch += [pltpu.VMEM((C, D), jnp.float32) for _ in range(NBUF)]
    scratch += [pltpu.SemaphoreType.DMA for _ in range(2 * NBUF)]

    @functools.partial(
        pl.kernel,
        mesh=mesh,
        out_type=jax.ShapeDtypeStruct((N, D), jnp.float32),
        scratch_types=scratch,
    )
    def gather_kernel(table_hbm, idx_hbm, out_hbm, idx_v, *rest):
        bufs = rest[:NBUF]
        gsem = rest[NBUF:2 * NBUF]
        ssem = rest[2 * NBUF:]
        wid = lax.axis_index("s") * NC + lax.axis_index("c")
        row0 = wid * R

        pltpu.sync_copy(idx_hbm.at[pl.ds(wid * NCH, NCH)], idx_v)

        def gather(g, b):
            return pltpu.make_async_copy(
                table_hbm.at[idx_v.at[g]], bufs[b], gsem[b])

        def store(g, b):
            return pltpu.make_async_copy(
                bufs[b], out_hbm.at[pl.ds(row0 + g * C, C)], ssem[b])

        gathers = [None] * NCH
        stores = [None] * NCH
        for g in range(min(NBUF, NCH)):
            gathers[g] = gather(g, g)
            gathers[g].start()
        for g in range(NCH):
            b = g % NBUF
            gathers[g].wait()
            stores[g] = store(g, b)
            stores[g].start()
            nxt = g + NBUF
            if nxt < NCH:
                stores[g].wait()
                gathers[nxt] = gather(nxt, b)
                gathers[nxt].start()
        for g in range(max(NCH - NBUF, 0), NCH):
            stores[g].wait()

    return gather_kernel


_gather = _make_gather()


@jax.jit
def kernel(seq_len, table):
    n, _ = table.shape
    shift = jnp.asarray(seq_len, jnp.int32) - n
    idx = jnp.clip(jnp.arange(n, dtype=jnp.int32) + shift, 0, n - 1)
    return _gather(table, idx.reshape(NW * NCH, C))
